# baseline (speedup 1.0000x reference)
import functools

import jax
import jax.numpy as jnp
from jax import lax
from jax.experimental import pallas as pl
from jax.experimental.pallas import tpu as pltpu
from jax.experimental.pallas import tpu_sc as plsc

_N, _M, _K = 100000, 26, 16
_B = 4096
_D = _M * _K
_MA = 16
_MB = _M - _MA
_DA = _MA * _K
_NC, _NS = 2, 16
_NW = _NC * _NS
_SPW = _B // _NW
_C = 4
_NCH = _SPW // _C
_IPW = _SPW * _M

_sc_mesh = plsc.VectorSubcoreMesh(core_axis_name="c", subcore_axis_name="s")
_sc_params = pltpu.CompilerParams(use_tc_tiling_on_sc=True)


def _permute(v, idx):
    dn = lax.GatherDimensionNumbers(
        offset_dims=(), collapsed_slice_dims=(0,), start_index_map=(0,))
    return lax.gather(v, idx[:, None], dn, slice_sizes=(1,),
                      mode=lax.GatherScatterMode.PROMISE_IN_BOUNDS)


def _lane_sum(acc, lanes):
    for sh in (1, 2, 4, 8):
        acc = acc + _permute(acc, jnp.bitwise_xor(lanes, sh))
    return acc


def _worker_id():
    return lax.axis_index("s") * _NC + lax.axis_index("c")


@functools.partial(
    pl.kernel,
    mesh=_sc_mesh,
    out_type=jax.ShapeDtypeStruct((_B,), jnp.float32),
    compiler_params=_sc_params,
    scratch_types=[
        pltpu.VMEM((_IPW,), jnp.int32),
        pltpu.VMEM((_SPW * _MA,), jnp.int32),
        pltpu.VMEM((2, _C * _MA, _DA), jnp.float32),
        pltpu.VMEM((_SPW * _K,), jnp.float32),
        pltpu.VMEM((_SPW,), jnp.float32),
        pltpu.SemaphoreType.DMA,
        pltpu.SemaphoreType.DMA,
    ],
)
def _ffm_low(x_hbm, taba_hbm, out_hbm, idx_v, idxa_v, bufs, accs_v, out_v,
             sem0, sem1):
    wid = _worker_id()
    lanes = lax.iota(jnp.int32, _K)
    pltpu.sync_copy(x_hbm.at[pl.ds(wid * _IPW, _IPW)], idx_v)

    def compact(r, _):
        idxa_v[pl.ds(r * _MA, _MA)] = idx_v[pl.ds(r * _M, _MA)]
        return 0

    lax.fori_loop(0, _SPW, compact, 0)
    sems = [sem0, sem1]
    rpc = _C * _MA

    def issue(c, b):
        pltpu.async_copy(
            taba_hbm.at[idxa_v.at[pl.ds(c * rpc, rpc)]], bufs.at[b], sems[b])

    def wait(b):
        pltpu.make_async_copy(
            taba_hbm.at[idxa_v.at[pl.ds(0, rpc)]], bufs.at[b], sems[b]).wait()

    def compute(c, b):
        buf = bufs.at[b]

        def sample_body(s, _):
            r0 = s * _MA
            acc = jnp.zeros((_K,), jnp.float32)
            for j1 in range(_MA):
                for j2 in range(j1 + 1, _MA):
                    acc = acc + (buf[r0 + j1, pl.ds(j2 * _K, _K)]
                                 * buf[r0 + j2, pl.ds(j1 * _K, _K)])
            acc = _lane_sum(acc, lanes)
            accs_v[pl.ds((c * _C + s) * _K, _K)] = acc
            return 0

        lax.fori_loop(0, _C, sample_body, 0)

    issue(0, 0)

    def outer(i, _):
        for b in range(2):
            c = i * 2 + b

            @pl.when(c + 1 < _NCH)
            def _():
                issue(c + 1, (b + 1) % 2)

            wait(b)
            compute(c, b)
        return 0

    lax.fori_loop(0, _NCH // 2, outer, 0)

    for g in range(_SPW // _K):
        tot = jnp.zeros((_K,), jnp.float32)
        for j in range(_K):
            row = accs_v[pl.ds((g * _K + j) * _K, _K)]
            tot = tot + jnp.where(lanes == j, row, 0.0)
        out_v[pl.ds(g * _K, _K)] = tot
    pltpu.sync_copy(out_v, out_hbm.at[pl.ds(wid * _SPW, _SPW)])


@functools.partial(
    pl.kernel,
    mesh=_sc_mesh,
    out_type=jax.ShapeDtypeStruct((_B,), jnp.float32),
    compiler_params=_sc_params,
    scratch_types=[
        pltpu.VMEM((_IPW,), jnp.int32),
        pltpu.VMEM((_SPW * _MB + 8,), jnp.int32),
        pltpu.VMEM((2, _C * _M, _DA), jnp.float32),
        pltpu.VMEM((2, _C * _MB, _DA), jnp.float32),
        pltpu.VMEM((_SPW * _K,), jnp.float32),
        pltpu.VMEM((_SPW,), jnp.float32),
        pltpu.SemaphoreType.DMA,
        pltpu.SemaphoreType.DMA,
    ],
)
def _ffm_high(x_hbm, tabb_hbm, taba_hbm, p1_hbm, out_hbm, idx_v, idxh_v,
              bufb, bufa, accs_v, out_v, sem0, sem1):
    wid = _worker_id()
    lanes = lax.iota(jnp.int32, _K)
    pltpu.sync_copy(x_hbm.at[pl.ds(wid * _IPW, _IPW)], idx_v)

    def compact(r, _):
        v = idx_v[pl.ds(r * _M + _MB, _K)]
        w = _permute(v, jnp.minimum(lanes + (_K - _MB), _K - 1))
        idxh_v[pl.ds(r * _MB, _K)] = w
        return 0

    lax.fori_loop(0, _SPW, compact, 0)
    sems = [sem0, sem1]
    rb = _C * _M
    ra = _C * _MB

    def issue(c, b):
        pltpu.async_copy(
            tabb_hbm.at[idx_v.at[pl.ds(c * rb, rb)]], bufb.at[b], sems[b])
        pltpu.async_copy(
            taba_hbm.at[idxh_v.at[pl.ds(c * ra, ra)]], bufa.at[b], sems[b])

    def wait(b):
        pltpu.make_async_copy(
            tabb_hbm.at[idx_v.at[pl.ds(0, rb)]], bufb.at[b], sems[b]).wait()
        pltpu.make_async_copy(
            taba_hbm.at[idxh_v.at[pl.ds(0, ra)]], bufa.at[b], sems[b]).wait()

    def compute(c, b):
        bb = bufb.at[b]
        ba = bufa.at[b]

        def sample_body(s, _):
            rb0 = s * _M
            ra0 = s * _MB
            acc = jnp.zeros((_K,), jnp.float32)
            for j1 in range(_MA):
                for j2 in range(_MA, _M):
                    acc = acc + (bb[rb0 + j1, pl.ds((j2 - _MA) * _K, _K)]
                                 * ba[ra0 + (j2 - _MA), pl.ds(j1 * _K, _K)])
            for j1 in range(_MA, _M):
                for j2 in range(j1 + 1, _M):
                    acc = acc + (bb[rb0 + j1, pl.ds((j2 - _MA) * _K, _K)]
                                 * bb[rb0 + j2, pl.ds((j1 - _MA) * _K, _K)])
            acc = _lane_sum(acc, lanes)
            accs_v[pl.ds((c * _C + s) * _K, _K)] = acc
            return 0

        lax.fori_loop(0, _C, sample_body, 0)

    issue(0, 0)

    def outer(i, _):
        for b in range(2):
            c = i * 2 + b

            @pl.when(c + 1 < _NCH)
            def _():
                issue(c + 1, (b + 1) % 2)

            wait(b)
            compute(c, b)
        return 0

    lax.fori_loop(0, _NCH // 2, outer, 0)

    pltpu.sync_copy(p1_hbm.at[pl.ds(wid * _SPW, _SPW)], out_v)
    for g in range(_SPW // _K):
        tot = out_v[pl.ds(g * _K, _K)]
        for j in range(_K):
            row = accs_v[pl.ds((g * _K + j) * _K, _K)]
            tot = tot + jnp.where(lanes == j, row, 0.0)
        out_v[pl.ds(g * _K, _K)] = 1.0 / (1.0 + jnp.exp(-tot))
    pltpu.sync_copy(out_v, out_hbm.at[pl.ds(wid * _SPW, _SPW)])


_NB = 4096
_NGRID = -(-_N // _NB)


def _transpose_a_body(ft_ref, out_ref):
    out_ref[...] = jnp.transpose(ft_ref[...], (1, 0))


_transpose_a = pl.pallas_call(
    _transpose_a_body,
    grid=(_NGRID,),
    in_specs=[pl.BlockSpec((_DA, _NB), lambda i: (0, i))],
    out_specs=pl.BlockSpec((_NB, _DA), lambda i: (i, 0)),
    out_shape=jax.ShapeDtypeStruct((_N, _DA), jnp.float32),
)


def _transpose_b_body(ft_ref, taba_ref, out_ref):
    del taba_ref
    out_ref[...] = jnp.transpose(ft_ref[...], (1, 0))


_transpose_b = pl.pallas_call(
    _transpose_b_body,
    grid=(_NGRID,),
    in_specs=[
        pl.BlockSpec((_DA, _NB), lambda i: (1, i)),
        pl.BlockSpec((8, 128), lambda i: (0, 0)),
    ],
    out_specs=pl.BlockSpec((_NB, _DA), lambda i: (i, 0)),
    out_shape=jax.ShapeDtypeStruct((_N, _DA), jnp.float32),
)


def kernel(x, field_embeddings):
    xf = x.reshape(-1).astype(jnp.int32)
    ft = jnp.transpose(field_embeddings, (1, 2, 0)).reshape(_D, _N)
    taba = _transpose_a(ft)
    p1 = _ffm_low(xf, taba)
    tabb = _transpose_b(ft, taba)
    return _ffm_high(xf, tabb, taba, p1)


# speedup vs baseline: 1.0905x; 1.0905x over previous
import functools

import jax
import jax.numpy as jnp
from jax import lax
from jax.experimental import pallas as pl
from jax.experimental.pallas import tpu as pltpu
from jax.experimental.pallas import tpu_sc as plsc

_N, _M, _K = 100000, 26, 16
_B = 4096
_D = _M * _K
_DP = 512
_NC, _NS = 2, 16
_NW = _NC * _NS
_SPW = _B // _NW
_C = 4
_NCH = _SPW // _C
_RPC = _C * _M
_IPW = _SPW * _M


def _permute(v, idx):
    dn = lax.GatherDimensionNumbers(
        offset_dims=(), collapsed_slice_dims=(0,), start_index_map=(0,))
    return lax.gather(v, idx[:, None], dn, slice_sizes=(1,),
                      mode=lax.GatherScatterMode.PROMISE_IN_BOUNDS)


@functools.partial(
    pl.kernel,
    mesh=plsc.VectorSubcoreMesh(core_axis_name="c", subcore_axis_name="s"),
    out_type=jax.ShapeDtypeStruct((_B,), jnp.float32),
    compiler_params=pltpu.CompilerParams(use_tc_tiling_on_sc=True),
    scratch_types=[
        pltpu.VMEM((_IPW,), jnp.int32),
        pltpu.VMEM((2, _RPC, _DP), jnp.float32),
        pltpu.VMEM((_SPW * _K,), jnp.float32),
        pltpu.VMEM((_SPW,), jnp.float32),
        pltpu.SemaphoreType.DMA,
        pltpu.SemaphoreType.DMA,
    ],
)
def _ffm_sc(x_hbm, tab_hbm, out_hbm, idx_v, bufs, accs_v, out_v, sem0, sem1):
    wid = lax.axis_index("s") * _NC + lax.axis_index("c")
    lanes = lax.iota(jnp.int32, _K)
    pltpu.sync_copy(x_hbm.at[pl.ds(wid * _IPW, _IPW)], idx_v)
    sems = [sem0, sem1]

    def issue(c, b):
        pltpu.async_copy(
            tab_hbm.at[idx_v.at[pl.ds(c * _RPC, _RPC)]], bufs.at[b], sems[b])

    def wait(b):
        pltpu.make_async_copy(
            tab_hbm.at[idx_v.at[pl.ds(0, _RPC)]], bufs.at[b], sems[b]).wait()

    def compute(c, b):
        buf = bufs.at[b]

        def sample_body(s, _):
            r0 = s * _M
            acc = jnp.zeros((_K,), jnp.float32)
            for j1 in range(_M):
                for j2 in range(j1 + 1, _M):
                    acc = acc + (buf[r0 + j1, pl.ds(j2 * _K, _K)]
                                 * buf[r0 + j2, pl.ds(j1 * _K, _K)])
            for sh in (1, 2, 4, 8):
                acc = acc + _permute(acc, jnp.bitwise_xor(lanes, sh))
            accs_v[pl.ds((c * _C + s) * _K, _K)] = acc
            return 0

        lax.fori_loop(0, _C, sample_body, 0)

    issue(0, 0)

    def outer(i, _):
        for b in range(2):
            c = i * 2 + b

            @pl.when(c + 1 < _NCH)
            def _():
                issue(c + 1, (b + 1) % 2)

            wait(b)
            compute(c, b)
        return 0

    lax.fori_loop(0, _NCH // 2, outer, 0)

    for g in range(_SPW // _K):
        tot = jnp.zeros((_K,), jnp.float32)
        for j in range(_K):
            row = accs_v[pl.ds((g * _K + j) * _K, _K)]
            tot = tot + jnp.where(lanes == j, row, 0.0)
        out_v[pl.ds(g * _K, _K)] = 1.0 / (1.0 + jnp.exp(-tot))
    pltpu.sync_copy(out_v, out_hbm.at[pl.ds(wid * _SPW, _SPW)])


_NB = 4096
_NGRID = -(-_N // _NB)


def _transpose_body(ft_ref, out_ref):
    out_ref[:, : _D] = jnp.transpose(ft_ref[...], (1, 0))


_transpose_tc = pl.pallas_call(
    _transpose_body,
    grid=(_NGRID,),
    in_specs=[pl.BlockSpec((_D, _NB), lambda i: (0, i))],
    out_specs=pl.BlockSpec((_NB, _DP), lambda i: (i, 0)),
    out_shape=jax.ShapeDtypeStruct((_N, _DP), jnp.float32),
)


def kernel(x, field_embeddings):
    xf = x.reshape(-1).astype(jnp.int32)
    ft = jnp.transpose(field_embeddings, (1, 2, 0)).reshape(_D, _N)
    tab = _transpose_tc(ft)
    return _ffm_sc(xf, tab)


# speedup vs baseline: 1.0998x; 1.0085x over previous
import functools

import jax
import jax.numpy as jnp
from jax import lax
from jax.experimental import pallas as pl
from jax.experimental.pallas import tpu as pltpu
from jax.experimental.pallas import tpu_sc as plsc

_N, _M, _K = 100000, 26, 16
_B = 4096
_D = _M * _K
_DP = 512
_NC, _NS = 2, 16
_NW = _NC * _NS
_SPW = _B // _NW
_C = 4
_NCH = _SPW // _C
_RPC = _C * _M
_IPW = _SPW * _M


def _permute(v, idx):
    dn = lax.GatherDimensionNumbers(
        offset_dims=(), collapsed_slice_dims=(0,), start_index_map=(0,))
    return lax.gather(v, idx[:, None], dn, slice_sizes=(1,),
                      mode=lax.GatherScatterMode.PROMISE_IN_BOUNDS)


@functools.partial(
    pl.kernel,
    mesh=plsc.VectorSubcoreMesh(core_axis_name="c", subcore_axis_name="s"),
    out_type=jax.ShapeDtypeStruct((_B,), jnp.float32),
    compiler_params=pltpu.CompilerParams(use_tc_tiling_on_sc=True),
    scratch_types=[
        pltpu.VMEM((_IPW,), jnp.int32),
        pltpu.VMEM((2, _RPC, _DP), jnp.float32),
        pltpu.VMEM((_SPW * _K,), jnp.float32),
        pltpu.VMEM((_SPW,), jnp.float32),
        pltpu.SemaphoreType.DMA,
        pltpu.SemaphoreType.DMA,
    ],
)
def _ffm_sc(x_hbm, tab_hbm, out_hbm, idx_v, bufs, accs_v, out_v, sem0, sem1):
    wid = lax.axis_index("s") * _NC + lax.axis_index("c")
    lanes = lax.iota(jnp.int32, _K)
    pltpu.sync_copy(x_hbm.at[pl.ds(wid * _IPW, _IPW)], idx_v)
    sems = [sem0, sem1]

    def issue(c, b):
        pltpu.async_copy(
            tab_hbm.at[idx_v.at[pl.ds(c * _RPC, _RPC)]], bufs.at[b], sems[b])

    def wait(b):
        pltpu.make_async_copy(
            tab_hbm.at[idx_v.at[pl.ds(0, _RPC)]], bufs.at[b], sems[b]).wait()

    def compute(c, b):
        buf = bufs.at[b]

        def sample_body(s, _):
            r0 = s * _M
            acc = jnp.zeros((_K,), jnp.float32)
            for j1 in range(_M):
                for j2 in range(j1 + 1, _M):
                    acc = acc + (buf[r0 + j1, pl.ds(j2 * _K, _K)]
                                 * buf[r0 + j2, pl.ds(j1 * _K, _K)])
            for sh in (1, 2, 4, 8):
                acc = acc + _permute(acc, jnp.bitwise_xor(lanes, sh))
            accs_v[pl.ds((c * _C + s) * _K, _K)] = acc
            return 0

        lax.fori_loop(0, _C, sample_body, 0)

    issue(0, 0)

    def outer(i, _):
        for b in range(2):
            c = i * 2 + b

            @pl.when(c + 1 < _NCH)
            def _():
                issue(c + 1, (b + 1) % 2)

            wait(b)
            compute(c, b)
        return 0

    lax.fori_loop(0, _NCH // 2, outer, 0)

    for g in range(_SPW // _K):
        tot = jnp.zeros((_K,), jnp.float32)
        for j in range(_K):
            row = accs_v[pl.ds((g * _K + j) * _K, _K)]
            tot = tot + jnp.where(lanes == j, row, 0.0)
        out_v[pl.ds(g * _K, _K)] = 1.0 / (1.0 + jnp.exp(-tot))
    pltpu.sync_copy(out_v, out_hbm.at[pl.ds(wid * _SPW, _SPW)])


_NB = 8192
_NGRID = -(-_N // _NB)


def _transpose_body(ft_ref, out_ref):
    out_ref[:, : _D] = jnp.transpose(ft_ref[...], (1, 0))


_transpose_tc = pl.pallas_call(
    _transpose_body,
    grid=(_NGRID,),
    in_specs=[pl.BlockSpec((_D, _NB), lambda i: (0, i))],
    out_specs=pl.BlockSpec((_NB, _DP), lambda i: (i, 0)),
    out_shape=jax.ShapeDtypeStruct((_N, _DP), jnp.float32),
    compiler_params=pltpu.CompilerParams(vmem_limit_bytes=100 * 1024 * 1024),
)


def kernel(x, field_embeddings):
    xf = x.reshape(-1).astype(jnp.int32)
    ft = jnp.transpose(field_embeddings, (1, 2, 0)).reshape(_D, _N)
    tab = _transpose_tc(ft)
    return _ffm_sc(xf, tab)
